# baseline (speedup 1.0000x reference)
import functools

import jax
import jax.numpy as jnp
from jax import lax
from jax.experimental import pallas as pl
from jax.experimental.pallas import tpu as pltpu
from jax.experimental.pallas import tpu_sc as plsc

_K = 51
_NW = 32


def _knn_body(prow_ref, pallT_ref, idx_ref, dpos_ref, dist_ref, *, K):
    prow = prow_ref[0]
    pallT = pallT_ref[0]
    Vt = prow.shape[0]
    V = pallT.shape[1]
    inner = lax.dot_general(prow.astype(jnp.bfloat16), pallT.astype(jnp.bfloat16),
                            (((1,), (0,)), ((), ())),
                            preferred_element_type=jnp.float32)
    sqr = jnp.sum(prow * prow, axis=1)[:, None]
    sqa = jnp.sum(pallT * pallT, axis=0)[None, :]
    dist = sqr - 2.0 * inner + sqa
    bits = lax.bitcast_convert_type(dist, jnp.int32)
    mint = jnp.int32(-2147483648)
    dist_ref[:, :] = jnp.where(bits < 0, (-1 - bits) ^ mint, bits)
    iot = lax.broadcasted_iota(jnp.int32, (Vt, V), 1)
    imax = jnp.int32(2147483647)

    def body(t, carry):
        d = dist_ref[:, :]
        m = jnp.min(d, axis=1, keepdims=True)
        a = jnp.min(jnp.where(d == m, iot, V), axis=1)
        idx_ref[0, pl.ds(t, 1), :] = (a + pl.program_id(0) * V)[None, :]
        onehot = iot == a[:, None]
        dist_ref[:, :] = jnp.where(onehot, imax, d)
        nb = jnp.stack(
            [jnp.sum(jnp.where(onehot, pallT[c][None, :], 0.0), axis=1)
             for c in range(3)], axis=1)
        dpos_ref[0, pl.ds(t, 1), :, :] = (nb - prow)[None]
        return carry

    lax.fori_loop(0, K, body, 0)


def _knn51(pos, K=_K):
    B, V, _ = pos.shape
    Vt = min(V, 256)
    grid = (B, V // Vt)
    posT = jnp.transpose(pos, (0, 2, 1))
    idx, dpos = pl.pallas_call(
        functools.partial(_knn_body, K=K),
        grid=grid,
        in_specs=[pl.BlockSpec((1, Vt, 3), lambda b, j: (b, j, 0)),
                  pl.BlockSpec((1, 3, V), lambda b, j: (b, 0, 0))],
        out_specs=[pl.BlockSpec((1, K, Vt), lambda b, j: (b, 0, j)),
                   pl.BlockSpec((1, K, Vt, 3), lambda b, j: (b, 0, j, 0))],
        out_shape=[jax.ShapeDtypeStruct((B, K, V), jnp.int32),
                   jax.ShapeDtypeStruct((B, K, V, 3), jnp.float32)],
        scratch_shapes=[pltpu.VMEM((Vt, V), jnp.int32)],
    )(pos, posT)
    return idx, dpos


@functools.partial(jax.jit, static_argnames=("fill", "cdma"))
def _sc_gather_call(table, idx, fill, cdma):
    M = idx.shape[0]
    D = table.shape[1]
    b_per_w = M // _NW
    n_fills = b_per_w // fill
    kf = fill // cdma
    mesh = plsc.VectorSubcoreMesh(core_axis_name="c", subcore_axis_name="s")

    @functools.partial(
        pl.kernel, mesh=mesh,
        out_type=jax.ShapeDtypeStruct((M, D), jnp.float32),
        scratch_types=[
            pltpu.VMEM((b_per_w,), jnp.int32),
            pltpu.VMEM((2, fill, D), jnp.float32),
            pltpu.SemaphoreType.DMA((2,)),
            pltpu.SemaphoreType.DMA((2,)),
        ],
    )
    def k(table_hbm, idx_hbm, out_hbm, idx_v, rows_v, gsem, ssem):
        wid = lax.axis_index("s") * 2 + lax.axis_index("c")
        base = wid * b_per_w
        pltpu.sync_copy(idx_hbm.at[pl.ds(base, b_per_w)], idx_v)

        def g_descs(f, buf):
            return [pltpu.make_async_copy(
                        table_hbm.at[idx_v.at[pl.ds(f * fill + j * cdma, cdma)]],
                        rows_v.at[buf, pl.ds(j * cdma, cdma)],
                        gsem.at[buf]) for j in range(kf)]

        def s_desc(f, buf):
            return pltpu.make_async_copy(
                rows_v.at[buf], out_hbm.at[pl.ds(base + f * fill, fill)],
                ssem.at[buf])

        for d in g_descs(0, 0):
            d.start()

        def body(f, carry):
            buf = lax.rem(f, 2)
            obuf = 1 - buf
            for d in g_descs(f, buf):
                d.wait()

            @pl.when(f >= 1)
            def _():
                s_desc(f - 1, obuf).wait()

            @pl.when(f + 1 < n_fills)
            def _():
                for d in g_descs(f + 1, obuf):
                    d.start()

            s_desc(f, buf).start()
            return carry

        lax.fori_loop(0, n_fills, body, 0)
        s_desc(n_fills - 1, lax.rem(n_fills - 1, 2)).wait()

    return k(table, idx)


def _sc_gather(table, idx):
    M = idx.shape[0]
    D = table.shape[1]
    cdma = 128 if D <= 256 else 64
    fill = 32768 // D
    step = _NW * fill
    M_pad = -(-M // step) * step
    if M_pad != M:
        idx = jnp.concatenate([idx, jnp.zeros((M_pad - M,), jnp.int32)])
    out = _sc_gather_call(table, idx, fill, cdma)
    return out[:M] if M_pad != M else out


def _bdot(a, b):
    return lax.dot_general(a.astype(jnp.bfloat16), b.astype(jnp.bfloat16),
                           (((a.ndim - 1,), (0,)), ((), ())),
                           preferred_element_type=jnp.float32)


def _mm_body(x_ref, w_ref, b_ref, o_ref, *, act):
    out = _bdot(x_ref[...], w_ref[...]) + b_ref[...]
    if act == 'relu':
        out = jnp.maximum(out, 0.0)
    o_ref[...] = out


def _mm_res_body(x_ref, w_ref, b_ref, r_ref, o_ref):
    o_ref[...] = _bdot(x_ref[...], w_ref[...]) + b_ref[...] + r_ref[...]


def _mm(x, w, b, act=None, res=None):
    R, Ci = x.shape
    Co = w.shape[1]
    Rt = min(R, 512)
    while R % Rt:
        Rt //= 2
    grid = (R // Rt,)
    b = b.reshape(1, Co)
    xs = pl.BlockSpec((Rt, Ci), lambda i: (i, 0))
    ws = pl.BlockSpec((Ci, Co), lambda i: (0, 0))
    bs = pl.BlockSpec((1, Co), lambda i: (0, 0))
    os_ = pl.BlockSpec((Rt, Co), lambda i: (i, 0))
    shape = jax.ShapeDtypeStruct((R, Co), jnp.float32)
    if res is None:
        return pl.pallas_call(
            functools.partial(_mm_body, act=act), grid=grid,
            in_specs=[xs, ws, bs], out_specs=os_, out_shape=shape,
        )(x, w, b)
    return pl.pallas_call(
        _mm_res_body, grid=grid,
        in_specs=[xs, ws, bs, os_], out_specs=os_, out_shape=shape,
    )(x, w, b, res)


def _bn_body(x_ref, g_ref, b_ref, o_ref):
    x = x_ref[...]
    mu = jnp.mean(x, axis=0, keepdims=True)
    var = jnp.mean((x - mu) ** 2, axis=0, keepdims=True)
    o_ref[...] = jnp.maximum(
        (x - mu) / jnp.sqrt(var + 1e-5) * g_ref[...] + b_ref[...],
        0.0)


def _bn_relu(x, g, b):
    R, C = x.shape
    g = g.reshape(1, C)
    b = b.reshape(1, C)
    return pl.pallas_call(
        _bn_body,
        in_specs=[pl.BlockSpec((R, C), lambda: (0, 0)),
                  pl.BlockSpec((1, C), lambda: (0, 0)),
                  pl.BlockSpec((1, C), lambda: (0, 0))],
        out_specs=pl.BlockSpec((R, C), lambda: (0, 0)),
        out_shape=jax.ShapeDtypeStruct((R, C), jnp.float32),
    )(x, g, b)


def _unit(x, axis):
    n = jnp.sqrt(jnp.sum(x * x, axis=axis, keepdims=True))
    return x / jnp.maximum(n, 1e-12)


def _conv_s_body(d_ref, dir_ref, o_ref, *, N):
    sdn = _unit(dir_ref[...], 0)

    def theta(nn):
        ndn = _unit(d_ref[:, nn, :], 1)
        return _bdot(ndn, sdn)

    acc = theta(0)
    for nn in range(1, N):
        acc = jnp.maximum(acc, theta(nn))
    o_ref[...] = jnp.maximum(acc, 0.0)


def _conv_surface_k(d, dirs):
    R, N, _ = d.shape
    kn = dirs.shape[1]
    Rt = min(R, 512)
    return pl.pallas_call(
        functools.partial(_conv_s_body, N=N), grid=(R // Rt,),
        in_specs=[pl.BlockSpec((Rt, N, 3), lambda i: (i, 0, 0)),
                  pl.BlockSpec((3, kn), lambda i: (0, 0))],
        out_specs=pl.BlockSpec((Rt, kn), lambda i: (i, 0)),
        out_shape=jax.ShapeDtypeStruct((R, kn), jnp.float32),
    )(d, dirs)


def _conv_l_body(d_ref, dir_ref, fs_ref, fc_ref, o_ref, *, N):
    sdn = _unit(dir_ref[...], 0)

    def term(nn):
        ndn = _unit(d_ref[:, nn, :], 1)
        th = jnp.maximum(_bdot(ndn, sdn), 0.0)
        return th * fs_ref[:, nn, :]

    acc = term(0)
    for nn in range(1, N):
        acc = jnp.maximum(acc, term(nn))
    o_ref[...] = fc_ref[...] + acc


def _conv_layer_k(d, dirs, fs_nb, fc):
    R, N, _ = d.shape
    o = dirs.shape[1]
    Rt = min(R, 32768 // o)
    while R % Rt:
        Rt //= 2
    return pl.pallas_call(
        functools.partial(_conv_l_body, N=N), grid=(R // Rt,),
        in_specs=[pl.BlockSpec((Rt, N, 3), lambda i: (i, 0, 0)),
                  pl.BlockSpec((3, o), lambda i: (0, 0)),
                  pl.BlockSpec((Rt, N, o), lambda i: (i, 0, 0)),
                  pl.BlockSpec((Rt, o), lambda i: (i, 0))],
        out_specs=pl.BlockSpec((Rt, o), lambda i: (i, 0)),
        out_shape=jax.ShapeDtypeStruct((R, o), jnp.float32),
    )(d, dirs, fs_nb, fc)


def _nbmax_body(x_ref, o_ref, *, N):
    acc = x_ref[:, 0, :]
    for nn in range(1, N):
        acc = jnp.maximum(acc, x_ref[:, nn, :])
    o_ref[...] = acc


def _nbmax(x):
    R, N, C = x.shape
    Rt = min(R, max(8, 131072 // (N * C)))
    while R % Rt:
        Rt //= 2
    return pl.pallas_call(
        functools.partial(_nbmax_body, N=N), grid=(R // Rt,),
        in_specs=[pl.BlockSpec((Rt, N, C), lambda i: (i, 0, 0))],
        out_specs=pl.BlockSpec((Rt, C), lambda i: (i, 0)),
        out_shape=jax.ShapeDtypeStruct((R, C), jnp.float32),
    )(x)


def _att_body(q_ref, kv_ref, pe_ref, w1_ref, b1_ref, w2_ref, b2_ref, o_ref, *, d):
    q = q_ref[...]
    w1 = w1_ref[...]
    b1 = b1_ref[...]
    w2 = w2_ref[...]
    b2 = b2_ref[...]
    N = kv_ref.shape[1]

    logits = []
    for nn in range(N):
        s = q - kv_ref[:, nn, :d] + pe_ref[:, nn, :]
        h = jnp.maximum(_bdot(s, w1) + b1, 0.0)
        logits.append(_bdot(h, w2) + b2)

    m = logits[0]
    for nn in range(1, N):
        m = jnp.maximum(m, logits[nn])
    es = [jnp.exp(l - m) for l in logits]
    z = es[0]
    for nn in range(1, N):
        z = z + es[nn]
    acc = jnp.zeros_like(q)
    for nn in range(N):
        acc = acc + (es[nn] / z) * (kv_ref[:, nn, d:] + pe_ref[:, nn, :])
    o_ref[...] = acc


def _att_tail(q, kvnb, pe, w1, b1, w2, b2):
    R, d = q.shape
    b1 = b1.reshape(1, d)
    b2 = b2.reshape(1, d)
    Rt = min(R, 65536 // d)
    while R % Rt:
        Rt //= 2
    return pl.pallas_call(
        functools.partial(_att_body, d=d), grid=(R // Rt,),
        in_specs=[pl.BlockSpec((Rt, d), lambda i: (i, 0)),
                  pl.BlockSpec((Rt, 16, 2 * d), lambda i: (i, 0, 0)),
                  pl.BlockSpec((Rt, 16, d), lambda i: (i, 0, 0)),
                  pl.BlockSpec((d, d), lambda i: (0, 0)),
                  pl.BlockSpec((1, d), lambda i: (0, 0)),
                  pl.BlockSpec((d, d), lambda i: (0, 0)),
                  pl.BlockSpec((1, d), lambda i: (0, 0))],
        out_specs=pl.BlockSpec((Rt, d), lambda i: (i, 0)),
        out_shape=jax.ShapeDtypeStruct((R, d), jnp.float32),
    )(q, kvnb, pe, w1, b1, w2, b2)


def _knn_bundle(v):
    B, V, _ = v.shape
    idxT, dposT = _knn51(v)
    idx = jnp.transpose(idxT, (0, 2, 1))
    d50 = jnp.transpose(dposT, (0, 2, 1, 3))[:, :, 1:51]
    d50 = d50.reshape(B * V, 50, 3)
    rel = (-d50[:, :16]).reshape(B * V * 16, 3)
    return idx, d50, rel


def _conv_layer(p, ii, d, fmap, out_ch):
    R = fmap.shape[0]
    N = ii.shape[0] // R
    fout = _mm(fmap, p['w'], p['b'])
    fc = fout[:, :out_ch]
    fs = fout[:, out_ch:]
    fs_nb = _sc_gather(fs, ii).reshape(R, N, out_ch)
    return _conv_layer_k(d, p['dir'], fs_nb, fc)


def _fusion_surface(p, knn, dim):
    idx, d50, _ = knn
    i50 = idx[:, :, 1:51].reshape(-1)
    fl = _bn_relu(_conv_surface_k(d50[:, :10], p['conv_l']['dir']),
                  p['bn_l']['g'], p['bn_l']['b'])
    fg = _bn_relu(_conv_surface_k(d50, p['conv_g0']['dir']),
                  p['bn_g0']['g'], p['bn_g0']['b'])
    fg = _bn_relu(_conv_layer(p['conv_g1'], i50, d50, fg, dim),
                  p['bn_g1']['g'], p['bn_g1']['b'])
    return jnp.concatenate([fl, fg], axis=1)


def _fusion(p, knn, inp, dim):
    idx, d50, _ = knn
    i10 = idx[:, :, 1:11].reshape(-1)
    i50 = idx[:, :, 1:51].reshape(-1)
    fl = _bn_relu(_conv_layer(p['conv_l'], i10, d50[:, :10], inp, dim),
                  p['bn_l']['g'], p['bn_l']['b'])
    fg = _bn_relu(_conv_layer(p['conv_g0'], i50, d50, inp, dim),
                  p['bn_g0']['g'], p['bn_g0']['b'])
    fg = _bn_relu(_conv_layer(p['conv_g1'], i50, d50, fg, dim),
                  p['bn_g1']['g'], p['bn_g1']['b'])
    return jnp.concatenate([fl, fg], axis=1)


def _transformer(p, knn, feat):
    idx, _, rel = knn
    R, dm = feat.shape
    i16 = idx[:, :, 1:17].reshape(-1)
    x = _mm(feat, p['start']['w'], p['start']['b'])
    wqkv = jnp.concatenate([p['q']['w'], p['k']['w'], p['v']['w']], axis=1)
    bqkv = jnp.concatenate([p['q']['b'], p['k']['b'], p['v']['b']])
    qkv = _mm(x, wqkv, bqkv)
    q = qkv[:, :dm]
    kv = qkv[:, dm:]
    kvnb = _sc_gather(kv, i16).reshape(R, 16, 2 * dm)
    pe = _mm(_mm(rel, p['pos1']['w'], p['pos1']['b'], act='relu'),
             p['pos2']['w'], p['pos2']['b']).reshape(R, 16, dm)
    agg = _att_tail(q, kvnb, pe, p['attn1']['w'], p['attn1']['b'],
                    p['attn2']['w'], p['attn2']['b'])
    return _mm(agg, p['end']['w'], p['end']['b'], res=feat)


def _pool(knn, v, fmap, rate=4):
    idx = knn[0]
    B, V, _ = idx.shape
    pool_num = V // rate
    i4 = idx[:, :pool_num, 1:5].reshape(-1)
    C = fmap.shape[1]
    nb = _sc_gather(fmap, i4).reshape(B * pool_num, 4, C)
    return v[:, :pool_num, :], _nbmax(nb)


def kernel(vertices, params):
    v = jnp.transpose(vertices, (0, 2, 1))
    B, V, _ = v.shape
    knn0 = _knn_bundle(v)

    fm0 = _fusion_surface(params['conv_0'], knn0, 128)
    fm0 = _mm(fm0, params['down0']['w'], params['down0']['b'], act='relu')
    fm0 = _transformer(params['att0'], knn0, fm0)
    fm1 = _fusion(params['conv_1'], knn0, fm0, 128)
    fm1 = _mm(fm1, params['down1']['w'], params['down1']['b'], act='relu')
    fm1 = _transformer(params['att1'], knn0, fm1)
    vp1, fp1 = _pool(knn0, v, fm1)

    knn1 = _knn_bundle(vp1)
    fm2 = _fusion(params['conv_2'], knn1, fp1, 128)
    fm2 = _transformer(params['att2'], knn1, fm2)
    fm3 = _fusion(params['conv_3'], knn1, fm2, 256)
    fm3 = _transformer(params['att3'], knn1, fm3)
    vp2, fp2 = _pool(knn1, vp1, fm3)

    knn2 = _knn_bundle(vp2)
    fm4 = _fusion(params['conv_4'], knn2, fp2, 512)
    fm4 = _mm(fm4, params['down2']['w'], params['down2']['b'], act='relu')
    fm4 = _transformer(params['att4'], knn2, fm4)

    V2 = vp2.shape[1]
    return _nbmax(fm4.reshape(B, V2, fm4.shape[1]))
